# baseline (speedup 1.0000x reference)
import functools

import jax
import jax.numpy as jnp
from jax.experimental import pallas as pl
from jax.experimental.pallas import tpu as pltpu

_MERGE_THRESHOLD = 0.85
_MIN_STRENGTH = 0.001
_STRENGTH_DECAY = 0.999
_WRITE_ALPHA = 0.25
_WRITE_BETA = 0.25
_BIG = 1 << 30


def _body(wk_ref, wv_ref, ws_ref, kb_ref, vb_ref, age_ref, st_ref,
          ko_ref, vo_ref, ageo_ref, sto_ref, slot_ref, sim_ref):
    bb, d, k = kb_ref.shape

    wk = wk_ref[...]
    wksq = jnp.sum(wk * wk, axis=1, keepdims=True)
    wk_nrm = jnp.sqrt(wksq) + 1e-6
    wkn3 = (wk / wk_nrm)[:, :, None]

    kb = kb_ref[...]
    dots = jnp.sum(kb * wkn3, axis=1)
    nsq = jnp.sum(kb * kb, axis=1)
    sim = dots / (jnp.sqrt(nsq) + 1e-6)

    best = jnp.max(sim, axis=1, keepdims=True)
    ki = jax.lax.broadcasted_iota(jnp.int32, (bb, k), 1)
    best_idx = jnp.min(jnp.where(sim == best, ki, _BIG), axis=1, keepdims=True)

    age = age_ref[...]
    st = st_ref[...]
    ascore = age + (1.0 - jnp.clip(st, 0.0, 1.0)) * 0.01
    amax = jnp.max(ascore, axis=1, keepdims=True)
    lru = jnp.min(jnp.where(ascore == amax, ki, _BIG), axis=1, keepdims=True)

    slot = jnp.where(best > _MERGE_THRESHOLD, best_idx, lru)
    at_slot = ki == slot

    ws = jnp.clip(ws_ref[...], 0.0, 1.0)
    ageo_ref[...] = jnp.where(at_slot, 0.0, age + 1.0)
    sdec = st * _STRENGTH_DECAY
    prev = jnp.sum(jnp.where(at_slot, sdec, 0.0), axis=1, keepdims=True)
    upd = jnp.clip(prev + ws * (1.0 - prev), _MIN_STRENGTH, 1.0)
    sto_ref[...] = jnp.where(at_slot, upd, sdec)

    sel = at_slot[:, None, :]

    alpha = _WRITE_ALPHA * ws
    oma = 1.0 - alpha
    dots_at = jnp.sum(jnp.where(at_slot, dots, 0.0), axis=1, keepdims=True)
    nsq_at = jnp.sum(jnp.where(at_slot, nsq, 0.0), axis=1, keepdims=True)
    dotw_at = dots_at * wk_nrm
    nk2 = oma * oma * nsq_at + 2.0 * alpha * oma * dotw_at + alpha * alpha * wksq
    rcp_k = 1.0 / (jnp.sqrt(nk2) + 1e-6)

    coef_k = (alpha * wk_nrm)[:, :, None]
    blend_k = (oma[:, :, None] * kb + coef_k * wkn3) * rcp_k[:, :, None]
    ko_ref[...] = jnp.where(sel, blend_k, kb)

    vb = vb_ref[...]
    wv3 = wv_ref[...][:, :, None]
    beta = _WRITE_BETA * ws
    blend_v = (1.0 - beta)[:, :, None] * vb + beta[:, :, None] * wv3
    vo_ref[...] = jnp.where(sel, blend_v, vb)

    slot_ref[...] = slot
    sim_ref[...] = best


@functools.partial(jax.jit, static_argnames=("bb", "interpret"))
def _run(write_key, write_val, write_strength, epi_keys, epi_vals, epi_age,
         epi_strength, bb=128, interpret=False):
    b, k, d = epi_keys.shape
    ekt = epi_keys.transpose(0, 2, 1)
    evt = epi_vals.transpose(0, 2, 1)

    grid = (b // bb,)
    rowd = pl.BlockSpec((bb, d), lambda i: (i, 0))
    rowk = pl.BlockSpec((bb, k), lambda i: (i, 0))
    row1 = pl.BlockSpec((bb, 1), lambda i: (i, 0))
    big = pl.BlockSpec((bb, d, k), lambda i: (i, 0, 0))

    outs = pl.pallas_call(
        _body,
        grid=grid,
        in_specs=[rowd, rowd, row1, big, big, rowk, rowk],
        out_specs=[big, big, rowk, rowk, row1, row1],
        out_shape=[
            jax.ShapeDtypeStruct((b, d, k), jnp.float32),
            jax.ShapeDtypeStruct((b, d, k), jnp.float32),
            jax.ShapeDtypeStruct((b, k), jnp.float32),
            jax.ShapeDtypeStruct((b, k), jnp.float32),
            jax.ShapeDtypeStruct((b, 1), jnp.int32),
            jax.ShapeDtypeStruct((b, 1), jnp.float32),
        ],
        compiler_params=pltpu.CompilerParams(
            dimension_semantics=("arbitrary",)),
        interpret=interpret,
    )(write_key, write_val, write_strength, ekt, evt, epi_age, epi_strength)

    ko, vo, ageo, sto, slot, sim = outs
    return (ko.transpose(0, 2, 1), vo.transpose(0, 2, 1), ageo, sto,
            slot.reshape(b), sim.reshape(b))


from jax import lax
from jax.experimental.pallas import tpu_sc as plsc

_NBUF = 4


def _rsqrt_sc(x):
    i = plsc.bitcast(x, jnp.int32)
    i = 0x5F3759DF - (i >> 1)
    y = plsc.bitcast(i, jnp.float32)
    for _ in range(3):
        y = y * (1.5 - 0.5 * x * y * y)
    return y


def _sqrt_sc(x):
    xs = jnp.maximum(x, 1e-30)
    return xs * _rsqrt_sc(xs)


def _sqrt_scalar(x):
    return _sqrt_sc(jnp.full((16,), x, jnp.float32))[0]


def _sc_row(i, kbuf, vbuf, abuf, sbuf, wkb, wvb, wsb, wknbuf, slotbuf, simbuf,
            d, k):
    iota = lax.iota(jnp.int32, 16)
    ng = k // 16
    nd = d // 16

    wkg = [plsc.load_gather(wkb, [iota + 16 * j, jnp.full((16,), i, jnp.int32)])
           for j in range(nd)]
    wksq = sum(jnp.sum(g * g, axis=0) for g in wkg)
    wk_nrm = _sqrt_scalar(wksq) + 1e-6
    for j in range(nd):
        wknbuf[pl.ds(16 * j, 16)] = wkg[j] / wk_nrm

    accd = [jnp.zeros((16,), jnp.float32) for _ in range(ng)]
    accs = [jnp.zeros((16,), jnp.float32) for _ in range(ng)]
    for j in range(nd):
        wv16 = wknbuf[pl.ds(16 * j, 16)]
        for t in range(16):
            dd = 16 * j + t
            w = wv16[t]
            for g in range(ng):
                v = kbuf[dd, pl.ds(16 * g, 16)]
                accd[g] = accd[g] + v * w
                accs[g] = accs[g] + v * v

    sims = [accd[g] / (_sqrt_sc(accs[g]) + 1e-6) for g in range(ng)]

    m = sims[0]
    for g in range(1, ng):
        m = jnp.maximum(m, sims[g])
    best = jnp.max(m, axis=0)
    big = jnp.full((16,), _BIG, jnp.int32)
    bidx = jnp.full((), _BIG, jnp.int32)
    for g in range(ng):
        cand = jnp.min(jnp.where(sims[g] == best, iota + 16 * g, big), axis=0)
        bidx = jnp.minimum(bidx, cand)

    ages = [abuf[pl.ds(16 * g, 16)] for g in range(ng)]
    sts = [sbuf[pl.ds(16 * g, 16)] for g in range(ng)]
    am = None
    asc = []
    for g in range(ng):
        a = ages[g] + (1.0 - jnp.clip(sts[g], 0.0, 1.0)) * 0.01
        asc.append(a)
        am = a if am is None else jnp.maximum(am, a)
    amax = jnp.max(am, axis=0)
    lidx = jnp.full((), _BIG, jnp.int32)
    for g in range(ng):
        cand = jnp.min(jnp.where(asc[g] == amax, iota + 16 * g, big), axis=0)
        lidx = jnp.minimum(lidx, cand)

    slot = jnp.where(best > _MERGE_THRESHOLD, bidx, lidx)
    irow = jnp.full((16,), i, jnp.int32)
    ws = jnp.clip(plsc.load_gather(wsb, [irow])[0], 0.0, 1.0)

    prev = jnp.zeros((), jnp.float32)
    for g in range(ng):
        msk = (iota + 16 * g) == slot
        sdec = sts[g] * _STRENGTH_DECAY
        prev = prev + jnp.sum(jnp.where(msk, sdec, 0.0), axis=0)
    upd = jnp.clip(prev + ws * (1.0 - prev), _MIN_STRENGTH, 1.0)
    for g in range(ng):
        msk = (iota + 16 * g) == slot
        sdec = sts[g] * _STRENGTH_DECAY
        abuf[pl.ds(16 * g, 16)] = jnp.where(msk, 0.0, ages[g] + 1.0)
        sbuf[pl.ds(16 * g, 16)] = jnp.where(msk, upd, sdec)

    alpha = _WRITE_ALPHA * ws
    beta = _WRITE_BETA * ws
    slotv = jnp.full((16,), slot, jnp.int32)
    newk = []
    nk2 = jnp.zeros((), jnp.float32)
    for j in range(nd):
        oldk = plsc.load_gather(kbuf, [iota + 16 * j, slotv])
        wkj = plsc.load_gather(wkb, [iota + 16 * j, irow])
        nk = (1.0 - alpha) * oldk + alpha * wkj
        newk.append(nk)
        nk2 = nk2 + jnp.sum(nk * nk, axis=0)
    nrm = _sqrt_scalar(nk2) + 1e-6
    for j in range(nd):
        plsc.store_scatter(kbuf, [iota + 16 * j, slotv], newk[j] / nrm)
    for j in range(nd):
        oldv = plsc.load_gather(vbuf, [iota + 16 * j, slotv])
        wvj = plsc.load_gather(wvb, [iota + 16 * j, irow])
        nv = (1.0 - beta) * oldv + beta * wvj
        plsc.store_scatter(vbuf, [iota + 16 * j, slotv], nv)

    lane0 = iota == 0
    plsc.store_scatter(slotbuf, [irow], jnp.full((16,), slot, jnp.int32),
                       mask=lane0)
    plsc.store_scatter(simbuf, [irow], jnp.full((16,), best, jnp.float32),
                       mask=lane0)


def _sc_impl(write_key, write_val, write_strength, epi_keys, epi_vals,
             epi_age, epi_strength):
    b, k, d = epi_keys.shape
    kd = k * d
    info = plsc.get_sparse_core_info()
    nw = info.num_cores * info.num_subcores
    rpw = b // nw

    ekf = epi_keys.transpose(0, 2, 1).reshape(b * d, k)
    evf = epi_vals.transpose(0, 2, 1).reshape(b * d, k)
    wkt = write_key.T
    wvt = write_val.T
    wsf = write_strength.reshape(b)
    agef = epi_age.reshape(b * k)
    stf = epi_strength.reshape(b * k)

    mesh = plsc.VectorSubcoreMesh(core_axis_name="c", subcore_axis_name="s")

    @functools.partial(
        pl.kernel,
        out_type=[
            jax.ShapeDtypeStruct((b * d, k), jnp.float32),
            jax.ShapeDtypeStruct((b * d, k), jnp.float32),
            jax.ShapeDtypeStruct((b * k,), jnp.float32),
            jax.ShapeDtypeStruct((b * k,), jnp.float32),
            jax.ShapeDtypeStruct((b,), jnp.int32),
            jax.ShapeDtypeStruct((b,), jnp.float32),
        ],
        mesh=mesh,
        compiler_params=pltpu.CompilerParams(needs_layout_passes=False),
        scratch_types=(
            [pltpu.VMEM((d, k), jnp.float32) for _ in range(_NBUF)]
            + [pltpu.VMEM((d, k), jnp.float32) for _ in range(_NBUF)]
            + [pltpu.VMEM((k,), jnp.float32) for _ in range(_NBUF)]
            + [pltpu.VMEM((k,), jnp.float32) for _ in range(_NBUF)]
            + [
                pltpu.VMEM((d, rpw), jnp.float32),
                pltpu.VMEM((d, rpw), jnp.float32),
                pltpu.VMEM((rpw,), jnp.float32),
                pltpu.VMEM((d,), jnp.float32),
                pltpu.VMEM((rpw,), jnp.int32),
                pltpu.VMEM((rpw,), jnp.float32),
            ]
        ),
    )
    def sck(wk_hbm, wv_hbm, ws_hbm, ek_hbm, ev_hbm, age_hbm, st_hbm,
            ko_hbm, vo_hbm, ageo_hbm, sto_hbm, slot_hbm, sim_hbm,
            *scr):
        kbufs = scr[0:_NBUF]
        vbufs = scr[_NBUF:2 * _NBUF]
        abufs = scr[2 * _NBUF:3 * _NBUF]
        sbufs = scr[3 * _NBUF:4 * _NBUF]
        wkb, wvb, wsb, wknbuf, slotbuf, simbuf = scr[4 * _NBUF:4 * _NBUF + 6]

        wid = lax.axis_index("s") * info.num_cores + lax.axis_index("c")
        b0 = wid * rpw

        pltpu.sync_copy(wk_hbm.at[:, pl.ds(b0, rpw)], wkb)
        pltpu.sync_copy(wv_hbm.at[:, pl.ds(b0, rpw)], wvb)
        pltpu.sync_copy(ws_hbm.at[pl.ds(b0, rpw)], wsb)

        @pl.loop(0, rpw)
        def _rows(i):
            r = 0
            pltpu.sync_copy(ek_hbm.at[pl.ds((b0 + i) * 64, 64)], kbufs[r])
            pltpu.sync_copy(ev_hbm.at[pl.ds((b0 + i) * 64, 64)], vbufs[r])
            pltpu.sync_copy(age_hbm.at[pl.ds((b0 + i) * k, k)], abufs[r])
            pltpu.sync_copy(st_hbm.at[pl.ds((b0 + i) * k, k)], sbufs[r])
            _sc_row(i, kbufs[r], vbufs[r], abufs[r], sbufs[r],
                    wkb, wvb, wsb, wknbuf, slotbuf, simbuf, d, k)
            pltpu.sync_copy(kbufs[r], ko_hbm.at[pl.ds((b0 + i) * 64, 64)])
            pltpu.sync_copy(vbufs[r], vo_hbm.at[pl.ds((b0 + i) * 64, 64)])
            pltpu.sync_copy(abufs[r], ageo_hbm.at[pl.ds((b0 + i) * k, k)])
            pltpu.sync_copy(sbufs[r], sto_hbm.at[pl.ds((b0 + i) * k, k)])

        pltpu.sync_copy(slotbuf, slot_hbm.at[pl.ds(b0, rpw)])
        pltpu.sync_copy(simbuf, sim_hbm.at[pl.ds(b0, rpw)])

    ko, vo, ageo, sto, slot, sim = sck(wkt, wvt, wsf, ekf, evf, agef, stf)
    return (ko.reshape(b, d, k).transpose(0, 2, 1),
            vo.reshape(b, d, k).transpose(0, 2, 1),
            ageo.reshape(b, k), sto.reshape(b, k), slot, sim)


def kernel(write_key, write_val, write_strength, epi_keys, epi_vals,
           epi_age, epi_strength):
    return _sc_impl(write_key, write_val, write_strength, epi_keys, epi_vals,
                    epi_age, epi_strength)


# speedup vs baseline: 1.6598x; 1.6598x over previous
import functools

import jax
import jax.numpy as jnp
from jax.experimental import pallas as pl
from jax.experimental.pallas import tpu as pltpu

_MERGE_THRESHOLD = 0.85
_MIN_STRENGTH = 0.001
_STRENGTH_DECAY = 0.999
_WRITE_ALPHA = 0.25
_WRITE_BETA = 0.25
_BIG = 1 << 30


def _body(wk_ref, wv_ref, ws_ref, kb_ref, vb_ref, age_ref, st_ref,
          ko_ref, vo_ref, ageo_ref, sto_ref, slot_ref, sim_ref):
    bb, d, k = kb_ref.shape

    wk = wk_ref[...]
    wksq = jnp.sum(wk * wk, axis=1, keepdims=True)
    wk_nrm = jnp.sqrt(wksq) + 1e-6
    wkn3 = (wk / wk_nrm)[:, :, None]

    kb = kb_ref[...]
    dots = jnp.sum(kb * wkn3, axis=1)
    nsq = jnp.sum(kb * kb, axis=1)
    sim = dots / (jnp.sqrt(nsq) + 1e-6)

    best = jnp.max(sim, axis=1, keepdims=True)
    ki = jax.lax.broadcasted_iota(jnp.int32, (bb, k), 1)
    best_idx = jnp.min(jnp.where(sim == best, ki, _BIG), axis=1, keepdims=True)

    age = age_ref[...]
    st = st_ref[...]
    ascore = age + (1.0 - jnp.clip(st, 0.0, 1.0)) * 0.01
    amax = jnp.max(ascore, axis=1, keepdims=True)
    lru = jnp.min(jnp.where(ascore == amax, ki, _BIG), axis=1, keepdims=True)

    slot = jnp.where(best > _MERGE_THRESHOLD, best_idx, lru)
    at_slot = ki == slot

    ws = jnp.clip(ws_ref[...], 0.0, 1.0)
    ageo_ref[...] = jnp.where(at_slot, 0.0, age + 1.0)
    sdec = st * _STRENGTH_DECAY
    prev = jnp.sum(jnp.where(at_slot, sdec, 0.0), axis=1, keepdims=True)
    upd = jnp.clip(prev + ws * (1.0 - prev), _MIN_STRENGTH, 1.0)
    sto_ref[...] = jnp.where(at_slot, upd, sdec)

    sel = at_slot[:, None, :]

    alpha = _WRITE_ALPHA * ws
    oma = 1.0 - alpha
    dots_at = jnp.sum(jnp.where(at_slot, dots, 0.0), axis=1, keepdims=True)
    nsq_at = jnp.sum(jnp.where(at_slot, nsq, 0.0), axis=1, keepdims=True)
    dotw_at = dots_at * wk_nrm
    nk2 = oma * oma * nsq_at + 2.0 * alpha * oma * dotw_at + alpha * alpha * wksq
    rcp_k = 1.0 / (jnp.sqrt(nk2) + 1e-6)

    coef_k = (alpha * wk_nrm)[:, :, None]
    blend_k = (oma[:, :, None] * kb + coef_k * wkn3) * rcp_k[:, :, None]
    ko_ref[...] = jnp.where(sel, blend_k, kb)

    vb = vb_ref[...]
    wv3 = wv_ref[...][:, :, None]
    beta = _WRITE_BETA * ws
    blend_v = (1.0 - beta)[:, :, None] * vb + beta[:, :, None] * wv3
    vo_ref[...] = jnp.where(sel, blend_v, vb)

    slot_ref[...] = slot
    sim_ref[...] = best


@functools.partial(jax.jit, static_argnames=("bb", "interpret"))
def _run(write_key, write_val, write_strength, epi_keys, epi_vals, epi_age,
         epi_strength, bb=128, interpret=False):
    b, k, d = epi_keys.shape
    ekt = epi_keys.transpose(0, 2, 1)
    evt = epi_vals.transpose(0, 2, 1)

    grid = (b // bb,)
    rowd = pl.BlockSpec((bb, d), lambda i: (i, 0))
    rowk = pl.BlockSpec((bb, k), lambda i: (i, 0))
    row1 = pl.BlockSpec((bb, 1), lambda i: (i, 0))
    big = pl.BlockSpec((bb, d, k), lambda i: (i, 0, 0))

    outs = pl.pallas_call(
        _body,
        grid=grid,
        in_specs=[rowd, rowd, row1, big, big, rowk, rowk],
        out_specs=[big, big, rowk, rowk, row1, row1],
        out_shape=[
            jax.ShapeDtypeStruct((b, d, k), jnp.float32),
            jax.ShapeDtypeStruct((b, d, k), jnp.float32),
            jax.ShapeDtypeStruct((b, k), jnp.float32),
            jax.ShapeDtypeStruct((b, k), jnp.float32),
            jax.ShapeDtypeStruct((b, 1), jnp.int32),
            jax.ShapeDtypeStruct((b, 1), jnp.float32),
        ],
        compiler_params=pltpu.CompilerParams(
            dimension_semantics=("arbitrary",)),
        interpret=interpret,
    )(write_key, write_val, write_strength, ekt, evt, epi_age, epi_strength)

    ko, vo, ageo, sto, slot, sim = outs
    return (ko.transpose(0, 2, 1), vo.transpose(0, 2, 1), ageo, sto,
            slot.reshape(b), sim.reshape(b))


from jax import lax
from jax.experimental.pallas import tpu_sc as plsc

_NBUF = 4


def _rsqrt_sc(x):
    i = plsc.bitcast(x, jnp.int32)
    i = 0x5F3759DF - (i >> 1)
    y = plsc.bitcast(i, jnp.float32)
    for _ in range(3):
        y = y * (1.5 - 0.5 * x * y * y)
    return y


def _sqrt_sc(x):
    xs = jnp.maximum(x, 1e-30)
    return xs * _rsqrt_sc(xs)


def _sqrt_scalar(x):
    return _sqrt_sc(jnp.full((16,), x, jnp.float32))[0]


def _sc_row(i, kbuf, vbuf, abuf, sbuf, wkb, wvb, wsb, wknbuf, slotbuf, simbuf,
            d, k):
    iota = lax.iota(jnp.int32, 16)
    ng = k // 16
    nd = d // 16

    wkg = [plsc.load_gather(wkb, [iota + 16 * j, jnp.full((16,), i, jnp.int32)])
           for j in range(nd)]
    wksq = sum(jnp.sum(g * g, axis=0) for g in wkg)
    wk_nrm = _sqrt_scalar(wksq) + 1e-6
    for j in range(nd):
        wknbuf[pl.ds(16 * j, 16)] = wkg[j] / wk_nrm

    accd = [jnp.zeros((16,), jnp.float32) for _ in range(ng)]
    accs = [jnp.zeros((16,), jnp.float32) for _ in range(ng)]
    for j in range(nd):
        wv16 = wknbuf[pl.ds(16 * j, 16)]
        for t in range(16):
            dd = 16 * j + t
            w = wv16[t]
            for g in range(ng):
                v = kbuf[dd, pl.ds(16 * g, 16)]
                accd[g] = accd[g] + v * w
                accs[g] = accs[g] + v * v

    sims = [accd[g] / (_sqrt_sc(accs[g]) + 1e-6) for g in range(ng)]

    m = sims[0]
    for g in range(1, ng):
        m = jnp.maximum(m, sims[g])
    best = jnp.max(m, axis=0)
    big = jnp.full((16,), _BIG, jnp.int32)
    bidx = jnp.full((), _BIG, jnp.int32)
    for g in range(ng):
        cand = jnp.min(jnp.where(sims[g] == best, iota + 16 * g, big), axis=0)
        bidx = jnp.minimum(bidx, cand)

    ages = [abuf[pl.ds(16 * g, 16)] for g in range(ng)]
    sts = [sbuf[pl.ds(16 * g, 16)] for g in range(ng)]
    am = None
    asc = []
    for g in range(ng):
        a = ages[g] + (1.0 - jnp.clip(sts[g], 0.0, 1.0)) * 0.01
        asc.append(a)
        am = a if am is None else jnp.maximum(am, a)
    amax = jnp.max(am, axis=0)
    lidx = jnp.full((), _BIG, jnp.int32)
    for g in range(ng):
        cand = jnp.min(jnp.where(asc[g] == amax, iota + 16 * g, big), axis=0)
        lidx = jnp.minimum(lidx, cand)

    slot = jnp.where(best > _MERGE_THRESHOLD, bidx, lidx)
    irow = jnp.full((16,), i, jnp.int32)
    ws = jnp.clip(plsc.load_gather(wsb, [irow])[0], 0.0, 1.0)

    prev = jnp.zeros((), jnp.float32)
    for g in range(ng):
        msk = (iota + 16 * g) == slot
        sdec = sts[g] * _STRENGTH_DECAY
        prev = prev + jnp.sum(jnp.where(msk, sdec, 0.0), axis=0)
    upd = jnp.clip(prev + ws * (1.0 - prev), _MIN_STRENGTH, 1.0)
    for g in range(ng):
        msk = (iota + 16 * g) == slot
        sdec = sts[g] * _STRENGTH_DECAY
        abuf[pl.ds(16 * g, 16)] = jnp.where(msk, 0.0, ages[g] + 1.0)
        sbuf[pl.ds(16 * g, 16)] = jnp.where(msk, upd, sdec)

    alpha = _WRITE_ALPHA * ws
    beta = _WRITE_BETA * ws
    slotv = jnp.full((16,), slot, jnp.int32)
    newk = []
    nk2 = jnp.zeros((), jnp.float32)
    for j in range(nd):
        oldk = plsc.load_gather(kbuf, [iota + 16 * j, slotv])
        wkj = plsc.load_gather(wkb, [iota + 16 * j, irow])
        nk = (1.0 - alpha) * oldk + alpha * wkj
        newk.append(nk)
        nk2 = nk2 + jnp.sum(nk * nk, axis=0)
    nrm = _sqrt_scalar(nk2) + 1e-6
    for j in range(nd):
        plsc.store_scatter(kbuf, [iota + 16 * j, slotv], newk[j] / nrm)
    for j in range(nd):
        oldv = plsc.load_gather(vbuf, [iota + 16 * j, slotv])
        wvj = plsc.load_gather(wvb, [iota + 16 * j, irow])
        nv = (1.0 - beta) * oldv + beta * wvj
        plsc.store_scatter(vbuf, [iota + 16 * j, slotv], nv)

    lane0 = iota == 0
    plsc.store_scatter(slotbuf, [irow], jnp.full((16,), slot, jnp.int32),
                       mask=lane0)
    plsc.store_scatter(simbuf, [irow], jnp.full((16,), best, jnp.float32),
                       mask=lane0)


def _sc_impl(write_key, write_val, write_strength, epi_keys, epi_vals,
             epi_age, epi_strength):
    b, k, d = epi_keys.shape
    kd = k * d
    info = plsc.get_sparse_core_info()
    nw = info.num_cores * info.num_subcores
    rpw = b // nw

    ekf = epi_keys.transpose(0, 2, 1).reshape(b * d, k)
    evf = epi_vals.transpose(0, 2, 1).reshape(b * d, k)
    wkt = write_key.T
    wvt = write_val.T
    wsf = write_strength.reshape(b)
    agef = epi_age.reshape(b * k)
    stf = epi_strength.reshape(b * k)

    mesh = plsc.VectorSubcoreMesh(core_axis_name="c", subcore_axis_name="s")

    @functools.partial(
        pl.kernel,
        out_type=[
            jax.ShapeDtypeStruct((b * d, k), jnp.float32),
            jax.ShapeDtypeStruct((b * d, k), jnp.float32),
            jax.ShapeDtypeStruct((b * k,), jnp.float32),
            jax.ShapeDtypeStruct((b * k,), jnp.float32),
            jax.ShapeDtypeStruct((b,), jnp.int32),
            jax.ShapeDtypeStruct((b,), jnp.float32),
        ],
        mesh=mesh,
        compiler_params=pltpu.CompilerParams(needs_layout_passes=False),
        scratch_types=(
            [pltpu.VMEM((d, k), jnp.float32) for _ in range(_NBUF)]
            + [pltpu.VMEM((d, k), jnp.float32) for _ in range(_NBUF)]
            + [pltpu.VMEM((k,), jnp.float32) for _ in range(_NBUF)]
            + [pltpu.VMEM((k,), jnp.float32) for _ in range(_NBUF)]
            + [
                pltpu.VMEM((d, rpw), jnp.float32),
                pltpu.VMEM((d, rpw), jnp.float32),
                pltpu.VMEM((rpw,), jnp.float32),
                pltpu.VMEM((d,), jnp.float32),
                pltpu.VMEM((rpw,), jnp.int32),
                pltpu.VMEM((rpw,), jnp.float32),
            ]
            + [pltpu.SemaphoreType.DMA] * (4 * 2)
        ),
    )
    def sck(wk_hbm, wv_hbm, ws_hbm, ek_hbm, ev_hbm, age_hbm, st_hbm,
            ko_hbm, vo_hbm, ageo_hbm, sto_hbm, slot_hbm, sim_hbm,
            *scr):
        kbufs = scr[0:_NBUF]
        vbufs = scr[_NBUF:2 * _NBUF]
        abufs = scr[2 * _NBUF:3 * _NBUF]
        sbufs = scr[3 * _NBUF:4 * _NBUF]
        wkb, wvb, wsb, wknbuf, slotbuf, simbuf = scr[4 * _NBUF:4 * _NBUF + 6]
        sems = scr[4 * _NBUF + 6:]
        ik_s, iv_s, ia_s, is_s = sems[0:2], sems[2:4], sems[4:6], sems[6:8]

        wid = lax.axis_index("s") * info.num_cores + lax.axis_index("c")
        b0 = wid * rpw

        pltpu.sync_copy(wk_hbm.at[:, pl.ds(b0, rpw)], wkb)
        pltpu.sync_copy(wv_hbm.at[:, pl.ds(b0, rpw)], wvb)
        pltpu.sync_copy(ws_hbm.at[pl.ds(b0, rpw)], wsb)

        def start_in(i, r):
            pltpu.async_copy(ek_hbm.at[pl.ds((b0 + i) * 64, 64)], kbufs[r],
                             ik_s[r])
            pltpu.async_copy(ev_hbm.at[pl.ds((b0 + i) * 64, 64)], vbufs[r],
                             iv_s[r])
            pltpu.async_copy(age_hbm.at[pl.ds((b0 + i) * k, k)], abufs[r],
                             ia_s[r])
            pltpu.async_copy(st_hbm.at[pl.ds((b0 + i) * k, k)], sbufs[r],
                             is_s[r])

        def wait_in(i, r):
            pltpu.make_async_copy(ek_hbm.at[pl.ds((b0 + i) * 64, 64)],
                                  kbufs[r], ik_s[r]).wait()
            pltpu.make_async_copy(ev_hbm.at[pl.ds((b0 + i) * 64, 64)],
                                  vbufs[r], iv_s[r]).wait()
            pltpu.make_async_copy(age_hbm.at[pl.ds((b0 + i) * k, k)],
                                  abufs[r], ia_s[r]).wait()
            pltpu.make_async_copy(st_hbm.at[pl.ds((b0 + i) * k, k)],
                                  sbufs[r], is_s[r]).wait()

        start_in(0, 0)

        @pl.loop(0, rpw // 2)
        def _rows(s_it):
            for r in range(2):
                i = s_it * 2 + r
                wait_in(i, r)

                @pl.when(i + 1 < rpw)
                def _():
                    start_in(i + 1, 1 - r)

                _sc_row(i, kbufs[r], vbufs[r], abufs[r], sbufs[r],
                        wkb, wvb, wsb, wknbuf, slotbuf, simbuf, d, k)
                pltpu.sync_copy(kbufs[r], ko_hbm.at[pl.ds((b0 + i) * 64, 64)])
                pltpu.sync_copy(vbufs[r], vo_hbm.at[pl.ds((b0 + i) * 64, 64)])
                pltpu.sync_copy(abufs[r], ageo_hbm.at[pl.ds((b0 + i) * k, k)])
                pltpu.sync_copy(sbufs[r], sto_hbm.at[pl.ds((b0 + i) * k, k)])

        pltpu.sync_copy(slotbuf, slot_hbm.at[pl.ds(b0, rpw)])
        pltpu.sync_copy(simbuf, sim_hbm.at[pl.ds(b0, rpw)])

    ko, vo, ageo, sto, slot, sim = sck(wkt, wvt, wsf, ekf, evf, agef, stf)
    return (ko.reshape(b, d, k).transpose(0, 2, 1),
            vo.reshape(b, d, k).transpose(0, 2, 1),
            ageo.reshape(b, k), sto.reshape(b, k), slot, sim)


def kernel(write_key, write_val, write_strength, epi_keys, epi_vals,
           epi_age, epi_strength):
    return _sc_impl(write_key, write_val, write_strength, epi_keys, epi_vals,
                    epi_age, epi_strength)
